# baseline (speedup 1.0000x reference)
import jax
import jax.numpy as jnp
from jax import lax
from jax.experimental import pallas as pl
from jax.experimental.pallas import tpu as pltpu
from jax.experimental.pallas import tpu_sc as plsc

N = 10000
E = 320000
IN_DIM = 257
D = 128
EO = 16
NO = 128

NW = 32
CB = 128
NCH = E // CB
NT = NCH // NW
TILES = 16
N_PAD = 10240
ROWS_PER_TILE = N_PAD // TILES


NEG_SCALE = -1.0

QN = N // 8
QP = N_PAD // 8


def _pre_body(x_ref, w1_ref, b1_ref, xa_ref, xb_ref):
    w1a = w1_ref[0:D, :] * NEG_SCALE
    w1b = w1_ref[D:2 * D, :] * NEG_SCALE
    b1v = b1_ref[...] * NEG_SCALE
    for j in range(8):
        xs = x_ref[:, pl.ds(j * D, D)]
        xa_ref[:, pl.ds(j * EO, EO)] = jnp.dot(
            xs, w1a, preferred_element_type=jnp.float32) + b1v
        xb_ref[:, pl.ds(j * EO, EO)] = jnp.dot(
            xs, w1b, preferred_element_type=jnp.float32)


def _pre_call(x2q, W1, b1r):
    return pl.pallas_call(
        _pre_body,
        grid=(1,),
        in_specs=[
            pl.BlockSpec((QN, 8 * D), lambda i: (0, 0)),
            pl.BlockSpec((IN_DIM, EO), lambda i: (0, 0)),
            pl.BlockSpec((1, EO), lambda i: (0, 0)),
        ],
        out_specs=[
            pl.BlockSpec((QN, 8 * EO), lambda i: (0, 0)),
            pl.BlockSpec((QN, 8 * EO), lambda i: (0, 0)),
        ],
        out_shape=[
            jax.ShapeDtypeStruct((QN, 8 * EO), jnp.float32),
            jax.ShapeDtypeStruct((QN, 8 * EO), jnp.float32),
        ],
    )(x2q, W1, b1r)


NBUF = 3
NOUTER = NT // NBUF


def _sc_body(xa_hbm, xb_hbm, ew_hbm, est_hbm, wrow_hbm, zeros_hbm,
             out_hbm, *scr):
    acc_p, acc_n, wr_v = scr[:3]
    idx2 = scr[3:3 + NBUF]
    ew_v = scr[3 + NBUF:3 + 2 * NBUF]
    ra = scr[3 + 2 * NBUF:3 + 3 * NBUF]
    rb = scr[3 + 3 * NBUF:3 + 4 * NBUF]
    msg = scr[3 + 4 * NBUF:3 + 5 * NBUF]
    sem_i = scr[3 + 5 * NBUF:3 + 6 * NBUF]
    sem_g = scr[3 + 6 * NBUF:3 + 7 * NBUF]
    sem_s = scr[3 + 7 * NBUF:3 + 8 * NBUF]

    c = lax.axis_index("c")
    s = lax.axis_index("s")
    w = c * TILES + s

    def fire_idx(slot, cc):
        pltpu.async_copy(est_hbm.at[cc], idx2[slot], sem_i[slot])
        pltpu.async_copy(ew_hbm.at[pl.ds(cc * CB, CB)], ew_v[slot],
                         sem_i[slot])

    def wait_idx(slot):
        pltpu.make_async_copy(est_hbm.at[0], idx2[slot], sem_i[slot]).wait()
        pltpu.make_async_copy(ew_hbm.at[pl.ds(0, CB)], ew_v[slot],
                              sem_i[slot]).wait()

    def fire_gather(slot):
        pltpu.async_copy(xa_hbm.at[idx2[slot].at[0]], ra[slot], sem_g[slot])
        pltpu.async_copy(xb_hbm.at[idx2[slot].at[1]], rb[slot], sem_g[slot])

    def wait_gather(slot):
        pltpu.make_async_copy(xa_hbm.at[idx2[slot].at[0]], ra[slot],
                              sem_g[slot]).wait()
        pltpu.make_async_copy(xb_hbm.at[idx2[slot].at[1]], rb[slot],
                              sem_g[slot]).wait()

    def compute(slot):
        for g in range(CB // EO):
            ewg = ew_v[slot][pl.ds(g * EO, EO)]
            ts = []
            for l in range(EO):
                i = g * EO + l
                ts.append(ra[slot][i, :] + rb[slot][i, :] + ewg[l] * wr)
            es = [jnp.exp(t) for t in ts]
            ms = [1.0 / (1.0 + e) for e in es]
            for l in range(EO):
                msg[slot][g * EO + l, :] = ms[l]
        pltpu.async_copy(msg[slot], acc_p.at[idx2[slot].at[1]], sem_s[slot],
                         add=True)
        pltpu.async_copy(msg[slot], acc_n.at[idx2[slot].at[0]], sem_s[slot],
                         add=True)

    def wait_scatter(slot):
        pltpu.make_async_copy(msg[slot], acc_p.at[idx2[slot].at[1]],
                              sem_s[slot]).wait()
        pltpu.make_async_copy(msg[slot], acc_n.at[idx2[slot].at[0]],
                              sem_s[slot]).wait()

    pltpu.sync_copy(zeros_hbm.at[pl.ds(s * ROWS_PER_TILE, ROWS_PER_TILE)],
                    acc_p.at[pl.ds(s * ROWS_PER_TILE, ROWS_PER_TILE)])
    pltpu.sync_copy(zeros_hbm.at[pl.ds(s * ROWS_PER_TILE, ROWS_PER_TILE)],
                    acc_n.at[pl.ds(s * ROWS_PER_TILE, ROWS_PER_TILE)])
    pltpu.sync_copy(wrow_hbm, wr_v)
    wr = wr_v[...]
    plsc.subcore_barrier()

    fire_idx(0, w)
    fire_idx(1, NW + w)
    wait_idx(0)
    fire_gather(0)

    pad_row = jnp.full((EO,), N, jnp.int32)
    for r in range(2):
        for g in range(CB // EO):
            idx2[2][r, pl.ds(g * EO, EO)] = pad_row
    pltpu.async_copy(msg[2], acc_p.at[idx2[2].at[1]], sem_s[2], add=True)
    pltpu.async_copy(msg[2], acc_n.at[idx2[2].at[0]], sem_s[2], add=True)

    def step(u, carry):
        for k in range(NBUF):
            t = u * NBUF + k
            s1 = (k + 1) % NBUF
            s2 = (k + 2) % NBUF
            wait_idx(s1)
            fire_gather(s1)
            wait_gather(k)
            compute(k)
            wait_scatter(s2)
            fire_idx(s2, jnp.minimum(t + 2, NT - 1) * NW + w)
        return carry

    lax.fori_loop(0, NOUTER, step, 0)

    wait_gather(0)
    wait_idx(1)
    wait_scatter(2)

    @pl.when(w < NCH - NT * NW)
    def _extra():
        cc = NT * NW + w
        pltpu.sync_copy(est_hbm.at[cc], idx2[0])
        pltpu.sync_copy(ew_hbm.at[pl.ds(cc * CB, CB)], ew_v[0])
        ga = pltpu.async_copy(xa_hbm.at[idx2[0].at[0]], ra[0], sem_g[0])
        gb = pltpu.async_copy(xb_hbm.at[idx2[0].at[1]], rb[0], sem_g[0])
        ga.wait()
        gb.wait()
        compute(0)
        wait_scatter(0)

    plsc.subcore_barrier()
    pltpu.sync_copy(acc_p.at[pl.ds(s * ROWS_PER_TILE, ROWS_PER_TILE)],
                    out_hbm.at[c, 0, pl.ds(s * ROWS_PER_TILE, ROWS_PER_TILE)])
    pltpu.sync_copy(acc_n.at[pl.ds(s * ROWS_PER_TILE, ROWS_PER_TILE)],
                    out_hbm.at[c, 1, pl.ds(s * ROWS_PER_TILE, ROWS_PER_TILE)])


def _sc_call(xa, xb, ew1, est3, wrow, zeros):
    mesh = plsc.VectorSubcoreMesh(core_axis_name="c", subcore_axis_name="s")
    scratch = [
        pltpu.VMEM_SHARED((N_PAD, EO), jnp.float32),
        pltpu.VMEM_SHARED((N_PAD, EO), jnp.float32),
        pltpu.VMEM((EO,), jnp.float32),
    ]
    scratch += [pltpu.VMEM((2, CB), jnp.int32) for _ in range(NBUF)]
    scratch += [pltpu.VMEM((CB,), jnp.float32) for _ in range(NBUF)]
    scratch += [pltpu.VMEM((CB, EO), jnp.float32) for _ in range(3 * NBUF)]
    scratch += [pltpu.SemaphoreType.DMA for _ in range(3 * NBUF)]
    fn = pl.kernel(
        _sc_body,
        out_type=jax.ShapeDtypeStruct((2, 2, N_PAD, EO), jnp.float32),
        mesh=mesh,
        scratch_types=scratch,
        compiler_params=pltpu.CompilerParams(use_tc_tiling_on_sc=False),
    )
    return fn(xa, xb, ew1, est3, wrow, zeros)


def _post_body(p_ref, w2_ref, b2_ref, o_ref):
    accv = (p_ref[0, 0] - p_ref[0, 1] + p_ref[1, 0] - p_ref[1, 1])[:QN]
    w2v = w2_ref[...]
    b2v = b2_ref[...]
    for j in range(8):
        h = jnp.dot(accv[:, j * EO:(j + 1) * EO], w2v,
                    preferred_element_type=jnp.float32)
        o_ref[:, pl.ds(j * NO, NO)] = jax.nn.sigmoid(h + b2v)


def _post_call(p4, W2, b2r):
    return pl.pallas_call(
        _post_body,
        grid=(1,),
        in_specs=[
            pl.BlockSpec((2, 2, QP, 8 * EO), lambda i: (0, 0, 0, 0)),
            pl.BlockSpec((EO, NO), lambda i: (0, 0)),
            pl.BlockSpec((1, NO), lambda i: (0, 0)),
        ],
        out_specs=pl.BlockSpec((QN, 8 * NO), lambda i: (0, 0)),
        out_shape=jax.ShapeDtypeStruct((QN, 8 * NO), jnp.float32),
    )(p4, W2, b2r)


def kernel(x, edge_src_target, edge_weight, W1, b1, W2, b2):
    x2q = x.reshape(QN, 8 * D)
    wrow = W1[2 * D] * NEG_SCALE

    xaq, xbq = _pre_call(x2q, W1, b1.reshape(1, EO))
    xa = xaq.reshape(N, EO)
    xb = xbq.reshape(N, EO)

    est3 = edge_src_target.reshape(2, NCH, CB).transpose(1, 0, 2)
    ew1 = edge_weight.reshape(E)
    zeros = jnp.zeros((N_PAD, EO), jnp.float32)

    partial = _sc_call(xa, xb, ew1, est3, wrow, zeros)

    p4 = partial.reshape(2, 2, QP, 8 * EO)
    out = _post_call(p4, W2, b2.reshape(1, NO))
    return out.reshape(1, N, NO)


# speedup vs baseline: 1.1199x; 1.1199x over previous
import jax
import jax.numpy as jnp
from jax import lax
from jax.experimental import pallas as pl
from jax.experimental.pallas import tpu as pltpu
from jax.experimental.pallas import tpu_sc as plsc

N = 10000
E = 320000
IN_DIM = 257
D = 128
EO = 16
NO = 128

NW = 32
CB = 128
NCH = E // CB
NT = NCH // NW
TILES = 16
N_PAD = 10240
ROWS_PER_TILE = N_PAD // TILES


NEG_SCALE = -1.0

QN = N // 8
QP = N_PAD // 8


def _pre_body(x_ref, w1_ref, b1_ref, xa_ref, xb_ref):
    w1a = w1_ref[0:D, :] * NEG_SCALE
    w1b = w1_ref[D:2 * D, :] * NEG_SCALE
    b1v = b1_ref[...] * NEG_SCALE
    for j in range(8):
        xs = x_ref[:, pl.ds(j * D, D)]
        xa_ref[:, pl.ds(j * EO, EO)] = jnp.dot(
            xs, w1a, preferred_element_type=jnp.float32) + b1v
        xb_ref[:, pl.ds(j * EO, EO)] = jnp.dot(
            xs, w1b, preferred_element_type=jnp.float32)


def _pre_call(x2q, W1, b1r):
    return pl.pallas_call(
        _pre_body,
        grid=(1,),
        in_specs=[
            pl.BlockSpec((QN, 8 * D), lambda i: (0, 0)),
            pl.BlockSpec((IN_DIM, EO), lambda i: (0, 0)),
            pl.BlockSpec((1, EO), lambda i: (0, 0)),
        ],
        out_specs=[
            pl.BlockSpec((QN, 8 * EO), lambda i: (0, 0)),
            pl.BlockSpec((QN, 8 * EO), lambda i: (0, 0)),
        ],
        out_shape=[
            jax.ShapeDtypeStruct((QN, 8 * EO), jnp.float32),
            jax.ShapeDtypeStruct((QN, 8 * EO), jnp.float32),
        ],
    )(x2q, W1, b1r)


NBUF = 3
NOUTER = NT // NBUF


def _sc_body(xa_hbm, xb_hbm, ew_hbm, est_hbm, wrow_hbm, zeros_hbm,
             out_hbm, *scr):
    acc_p, acc_n, wr_v = scr[:3]
    idx2 = scr[3:3 + NBUF]
    ew_v = scr[3 + NBUF:3 + 2 * NBUF]
    ra = scr[3 + 2 * NBUF:3 + 3 * NBUF]
    rb = scr[3 + 3 * NBUF:3 + 4 * NBUF]
    msg = scr[3 + 4 * NBUF:3 + 5 * NBUF]
    sem_i = scr[3 + 5 * NBUF:3 + 6 * NBUF]
    sem_g = scr[3 + 6 * NBUF:3 + 7 * NBUF]

    c = lax.axis_index("c")
    s = lax.axis_index("s")
    w = c * TILES + s

    def fire_idx(slot, cc):
        pltpu.async_copy(est_hbm.at[cc], idx2[slot], sem_i[slot])
        pltpu.async_copy(ew_hbm.at[pl.ds(cc * CB, CB)], ew_v[slot],
                         sem_i[slot])

    def wait_idx(slot):
        pltpu.make_async_copy(est_hbm.at[0], idx2[slot], sem_i[slot]).wait()
        pltpu.make_async_copy(ew_hbm.at[pl.ds(0, CB)], ew_v[slot],
                              sem_i[slot]).wait()

    def fire_gather(slot):
        pltpu.async_copy(xa_hbm.at[idx2[slot].at[0]], ra[slot], sem_g[slot])
        pltpu.async_copy(xb_hbm.at[idx2[slot].at[1]], rb[slot], sem_g[slot])

    def wait_gather(slot):
        pltpu.make_async_copy(xa_hbm.at[idx2[slot].at[0]], ra[slot],
                              sem_g[slot]).wait()
        pltpu.make_async_copy(xb_hbm.at[idx2[slot].at[1]], rb[slot],
                              sem_g[slot]).wait()

    def compute(slot):
        for g in range(CB // EO):
            ewg = ew_v[slot][pl.ds(g * EO, EO)]
            ts = []
            for l in range(EO):
                i = g * EO + l
                ts.append(ra[slot][i, :] + rb[slot][i, :] + ewg[l] * wr)
            es = [jnp.exp(t) for t in ts]
            ms = [1.0 / (1.0 + e) for e in es]
            for l in range(EO):
                msg[slot][g * EO + l, :] = ms[l]
        pltpu.sync_copy(msg[slot], acc_p.at[idx2[slot].at[1]], add=True)
        pltpu.sync_copy(msg[slot], acc_n.at[idx2[slot].at[0]], add=True)

    pltpu.sync_copy(zeros_hbm.at[pl.ds(s * ROWS_PER_TILE, ROWS_PER_TILE)],
                    acc_p.at[pl.ds(s * ROWS_PER_TILE, ROWS_PER_TILE)])
    pltpu.sync_copy(zeros_hbm.at[pl.ds(s * ROWS_PER_TILE, ROWS_PER_TILE)],
                    acc_n.at[pl.ds(s * ROWS_PER_TILE, ROWS_PER_TILE)])
    pltpu.sync_copy(wrow_hbm, wr_v)
    wr = wr_v[...]
    plsc.subcore_barrier()

    fire_idx(0, w)
    fire_idx(1, NW + w)
    wait_idx(0)
    fire_gather(0)

    def step(u, carry):
        for k in range(NBUF):
            t = u * NBUF + k
            s1 = (k + 1) % NBUF
            s2 = (k + 2) % NBUF
            wait_idx(s1)
            fire_gather(s1)
            wait_gather(k)
            fire_idx(s2, jnp.minimum(t + 2, NT - 1) * NW + w)
            compute(k)
        return carry

    lax.fori_loop(0, NOUTER, step, 0)

    wait_gather(0)
    wait_idx(1)

    @pl.when(w < NCH - NT * NW)
    def _extra():
        cc = NT * NW + w
        pltpu.sync_copy(est_hbm.at[cc], idx2[0])
        pltpu.sync_copy(ew_hbm.at[pl.ds(cc * CB, CB)], ew_v[0])
        ga = pltpu.async_copy(xa_hbm.at[idx2[0].at[0]], ra[0], sem_g[0])
        gb = pltpu.async_copy(xb_hbm.at[idx2[0].at[1]], rb[0], sem_g[0])
        ga.wait()
        gb.wait()
        compute(0)

    plsc.subcore_barrier()
    pltpu.sync_copy(acc_p.at[pl.ds(s * ROWS_PER_TILE, ROWS_PER_TILE)],
                    out_hbm.at[c, 0, pl.ds(s * ROWS_PER_TILE, ROWS_PER_TILE)])
    pltpu.sync_copy(acc_n.at[pl.ds(s * ROWS_PER_TILE, ROWS_PER_TILE)],
                    out_hbm.at[c, 1, pl.ds(s * ROWS_PER_TILE, ROWS_PER_TILE)])


def _sc_call(xa, xb, ew1, est3, wrow, zeros):
    mesh = plsc.VectorSubcoreMesh(core_axis_name="c", subcore_axis_name="s")
    scratch = [
        pltpu.VMEM_SHARED((N_PAD, EO), jnp.float32),
        pltpu.VMEM_SHARED((N_PAD, EO), jnp.float32),
        pltpu.VMEM((EO,), jnp.float32),
    ]
    scratch += [pltpu.VMEM((2, CB), jnp.int32) for _ in range(NBUF)]
    scratch += [pltpu.VMEM((CB,), jnp.float32) for _ in range(NBUF)]
    scratch += [pltpu.VMEM((CB, EO), jnp.float32) for _ in range(3 * NBUF)]
    scratch += [pltpu.SemaphoreType.DMA for _ in range(2 * NBUF)]
    fn = pl.kernel(
        _sc_body,
        out_type=jax.ShapeDtypeStruct((2, 2, N_PAD, EO), jnp.float32),
        mesh=mesh,
        scratch_types=scratch,
        compiler_params=pltpu.CompilerParams(use_tc_tiling_on_sc=False),
    )
    return fn(xa, xb, ew1, est3, wrow, zeros)


def _post_body(p_ref, w2_ref, b2_ref, o_ref):
    accv = (p_ref[0, 0] - p_ref[0, 1] + p_ref[1, 0] - p_ref[1, 1])[:QN]
    w2v = w2_ref[...]
    b2v = b2_ref[...]
    for j in range(8):
        h = jnp.dot(accv[:, j * EO:(j + 1) * EO], w2v,
                    preferred_element_type=jnp.float32)
        o_ref[:, pl.ds(j * NO, NO)] = jax.nn.sigmoid(h + b2v)


def _post_call(p4, W2, b2r):
    return pl.pallas_call(
        _post_body,
        grid=(1,),
        in_specs=[
            pl.BlockSpec((2, 2, QP, 8 * EO), lambda i: (0, 0, 0, 0)),
            pl.BlockSpec((EO, NO), lambda i: (0, 0)),
            pl.BlockSpec((1, NO), lambda i: (0, 0)),
        ],
        out_specs=pl.BlockSpec((QN, 8 * NO), lambda i: (0, 0)),
        out_shape=jax.ShapeDtypeStruct((QN, 8 * NO), jnp.float32),
    )(p4, W2, b2r)


def kernel(x, edge_src_target, edge_weight, W1, b1, W2, b2):
    x2q = x.reshape(QN, 8 * D)
    wrow = W1[2 * D] * NEG_SCALE

    xaq, xbq = _pre_call(x2q, W1, b1.reshape(1, EO))
    xa = xaq.reshape(N, EO)
    xb = xbq.reshape(N, EO)

    est3 = edge_src_target.reshape(2, NCH, CB).transpose(1, 0, 2)
    ew1 = edge_weight.reshape(E)
    zeros = jnp.zeros((N_PAD, EO), jnp.float32)

    partial = _sc_call(xa, xb, ew1, est3, wrow, zeros)

    p4 = partial.reshape(2, 2, QP, 8 * EO)
    out = _post_call(p4, W2, b2.reshape(1, NO))
    return out.reshape(1, N, NO)
